# baseline (speedup 1.0000x reference)
import functools

import jax
import jax.numpy as jnp
import numpy as np
from jax import lax
from jax.experimental import pallas as pl
from jax.experimental.pallas import tpu as pltpu
from jax.experimental.pallas import tpu_sc as plsc

_X_TILES = 4
_Y_TILES = 4
_NWORKERS = 32


@functools.lru_cache(maxsize=None)
def _perm_table(B):
    with jax.ensure_compile_time_eval():
        u = jax.random.uniform(jax.random.key(42), (B, _Y_TILES * _X_TILES))
        perm = np.asarray(jnp.argsort(u, axis=-1))
    return perm.astype(np.int32)


def kernel(image):
    B, C, H, W = image.shape
    hs, ws = _Y_TILES, _X_TILES
    h, w = H // hs, W // ws
    ntiles = hs * ws
    nslab = B * ntiles
    slabs_per_worker = nslab // _NWORKERS

    src_tile = jnp.asarray(_perm_table(B).reshape(_NWORKERS, slabs_per_worker))
    x6 = image.reshape(B, C, hs, h, ws, w)

    mesh = plsc.VectorSubcoreMesh(core_axis_name="core", subcore_axis_name="subcore")

    @functools.partial(
        pl.kernel,
        out_type=jax.ShapeDtypeStruct((B, C, hs, h, ws, w), image.dtype),
        mesh=mesh,
        compiler_params=pltpu.CompilerParams(use_tc_tiling_on_sc=False),
        scratch_types=[
            pltpu.VMEM((slabs_per_worker + 16,), jnp.int32),
            pltpu.VMEM((C, h, w), jnp.float32),
            pltpu.VMEM((C, h, w), jnp.float32),
            pltpu.SemaphoreType.DMA,
            pltpu.SemaphoreType.DMA,
            pltpu.SemaphoreType.DMA,
            pltpu.SemaphoreType.DMA,
            pltpu.SemaphoreType.DMA,
        ],
    )
    def move_slabs(x_hbm, t_hbm, o_hbm, t_vmem, buf_a, buf_b, isem, gs_a, gs_b, ss_a, ss_b):
        wid = lax.axis_index("subcore") * 2 + lax.axis_index("core")
        bufs = (buf_a, buf_b)
        gsems = (gs_a, gs_b)
        ssems = (ss_a, ss_b)

        pltpu.async_copy(t_hbm.at[wid], t_vmem.at[pl.ds(0, slabs_per_worker)], isem).wait()
        base = wid * slabs_per_worker

        def out_slab(k):
            g = base + k
            b = g // ntiles
            t = g % ntiles
            return o_hbm.at[b, :, t // ws, :, t % ws, :]

        def src_slab(k):
            g = base + k
            b = g // ntiles
            s = t_vmem[pl.ds(k, 16)][0]
            return x_hbm.at[b, :, s // ws, :, s % ws, :]

        @pl.loop(0, slabs_per_worker, step=2)
        def _(k0):
            for s in range(2):
                k = k0 + s

                @pl.when(k >= 2)
                def _():
                    pltpu.make_async_copy(bufs[s], out_slab(k - 2), ssems[s]).wait()

                pltpu.async_copy(src_slab(k), bufs[s], gsems[s]).wait()
                pltpu.async_copy(bufs[s], out_slab(k), ssems[s])

        for s in range(2):
            k_last = slabs_per_worker - 2 + s
            pltpu.make_async_copy(bufs[s], out_slab(k_last), ssems[s]).wait()

    out = move_slabs(x6, src_tile)
    return out.reshape(B, C, H, W)
